# Initial kernel scaffold; baseline (speedup 1.0000x reference)
import jax
import jax.numpy as jnp
from jax.experimental import pallas as pl


def kernel(x, edge_index, W1, b1, W2, b2):
    raise NotImplementedError("write your pallas kernel here")


# speedup vs baseline: 10.7757x; 10.7757x over previous
import functools

import jax
import jax.numpy as jnp
from jax import lax
from jax.experimental import pallas as pl
from jax.experimental.pallas import tpu as pltpu
from jax.experimental.pallas import tpu_sc as plsc

N = 10000
E = 320000
D = 128
K = 10
ALPHA = 0.1

NP = 10240
EP = 327680
NTILE = 16
ET = EP // NTILE
B = 128
NBLK = ET // B
NT = NP // NTILE
F = 64
NV = F // 16
CH = 64
NCH = NT // CH


def _fc_relu_body(x_ref, w_ref, b_ref, o_ref):
    o_ref[...] = jnp.maximum(
        jnp.dot(x_ref[...], w_ref[...], preferred_element_type=jnp.float32)
        + b_ref[...], 0.0)


def _fc_body(x_ref, w_ref, b_ref, o_ref):
    o_ref[...] = (
        jnp.dot(x_ref[...], w_ref[...], preferred_element_type=jnp.float32)
        + b_ref[...])


def _fc(x, w, b, relu):
    m = x.shape[0]
    blk = 1000
    return pl.pallas_call(
        _fc_relu_body if relu else _fc_body,
        grid=(m // blk,),
        in_specs=[
            pl.BlockSpec((blk, D), lambda i: (i, 0)),
            pl.BlockSpec((D, D), lambda i: (0, 0)),
            pl.BlockSpec((1, D), lambda i: (0, 0)),
        ],
        out_specs=pl.BlockSpec((blk, D), lambda i: (i, 0)),
        out_shape=jax.ShapeDtypeStruct((m, D), jnp.float32),
    )(x, w, b)


_mesh = plsc.VectorSubcoreMesh(core_axis_name="c", subcore_axis_name="s")


@functools.partial(
    pl.kernel,
    mesh=_mesh,
    compiler_params=pltpu.CompilerParams(use_tc_tiling_on_sc=False),
    out_type=(
        jax.ShapeDtypeStruct((2 * NP, F), jnp.float32),
        jax.ShapeDtypeStruct((2 * NP, F), jnp.float32),
    ),
    scratch_types=[
        pltpu.VMEM((NBLK, B), jnp.int32),
        pltpu.VMEM((NBLK, B), jnp.int32),
        pltpu.VMEM((B, F), jnp.float32),
        pltpu.VMEM((B, F), jnp.float32),
        pltpu.VMEM((CH, F), jnp.float32),
        pltpu.VMEM((CH, F), jnp.float32),
        pltpu.VMEM((CH, F), jnp.float32),
        pltpu.VMEM((CH, F), jnp.float32),
        pltpu.VMEM((NT,), jnp.float32),
        pltpu.VMEM((NT,), jnp.float32),
        pltpu.VMEM((NT,), jnp.float32),
        pltpu.VMEM((B,), jnp.float32),
        pltpu.VMEM_SHARED((NP, F), jnp.float32),
        pltpu.VMEM_SHARED((NP,), jnp.float32),
        pltpu.SemaphoreType.DMA,
    ],
)
def _appnp_sc(h_hbm, rowb_hbm, col_hbm, out_hbm, g_hbm,
              row_v, col_v, gbuf0, gbuf1, abuf, ubuf, gchk, zbuf,
              coef_v, sqd_v, sv_v, ones_v, agg_sh, deg_sh, sem):
    c = lax.axis_index("c")
    s = lax.axis_index("s")
    base = s * NT
    hrow = c * NP + base

    pltpu.sync_copy(rowb_hbm.at[c, s], row_v)
    pltpu.sync_copy(col_hbm.at[s], col_v)

    zf16 = jnp.zeros((16,), jnp.float32)

    def _zero_zbuf(i, _):
        r = i // NV
        f = (i % NV) * 16
        zbuf[r, pl.ds(f, 16)] = zf16
        return 0
    lax.fori_loop(0, CH * NV, _zero_zbuf, 0)

    def _zero_sv(i, _):
        sv_v[pl.ds(i * 16, 16)] = zf16
        return 0
    lax.fori_loop(0, NT // 16, _zero_sv, 0)

    def _ones(i, _):
        ones_v[pl.ds(i * 16, 16)] = jnp.ones((16,), jnp.float32)
        return 0
    lax.fori_loop(0, B // 16, _ones, 0)

    pltpu.sync_copy(sv_v, deg_sh.at[pl.ds(base, NT)])

    def _zero_agg(i, _):
        pltpu.sync_copy(zbuf, agg_sh.at[pl.ds(base + i * CH, CH)])
        return 0
    lax.fori_loop(0, NCH, _zero_agg, 0)

    plsc.subcore_barrier()

    def _deg_blk(j, _):
        pltpu.sync_copy(ones_v, deg_sh.at[col_v.at[j]], add=True)
        return 0
    lax.fori_loop(0, NBLK, _deg_blk, 0)

    plsc.subcore_barrier()

    pltpu.sync_copy(deg_sh.at[pl.ds(base, NT)], sv_v)

    def _coef(i, _):
        ds16 = pl.ds(i * 16, 16)
        d = sv_v[ds16] + 1.0
        y = 0.5 * (d + 1.0)
        for _it in range(20):
            y = 0.5 * (y + d / y)
        coef_v[ds16] = (1.0 - ALPHA) / d
        sqd_v[ds16] = y
        sv_v[ds16] = 1.0 / y
        return 0
    lax.fori_loop(0, NT // 16, _coef, 0)

    def _u0(i, _):
        rs = i * CH
        pltpu.sync_copy(h_hbm.at[pl.ds(hrow + rs, CH)], ubuf)

        def _sc16(g, _2):
            sv16 = sv_v[pl.ds(rs + g * 16, 16)]
            for r in range(16):
                sv = sv16[r]
                for f in range(NV):
                    ds16 = pl.ds(f * 16, 16)
                    val = ubuf[g * 16 + r, ds16] * sv
                    ubuf[g * 16 + r, ds16] = val
                    gchk[g * 16 + r, ds16] = val * ALPHA
            return 0
        lax.fori_loop(0, CH // 16, _sc16, 0)

        pltpu.sync_copy(ubuf, out_hbm.at[pl.ds(hrow + rs, CH)])
        pltpu.sync_copy(gchk, g_hbm.at[pl.ds(hrow + rs, CH)])
        return 0
    lax.fori_loop(0, NCH, _u0, 0)

    plsc.subcore_barrier()

    def _step(_k, _):
        def _blk(j, _2):
            pltpu.async_copy(out_hbm.at[row_v.at[2 * j]], gbuf0, sem).wait()
            pltpu.sync_copy(gbuf0, agg_sh.at[col_v.at[2 * j]], add=True)
            pltpu.async_copy(out_hbm.at[row_v.at[2 * j + 1]], gbuf1, sem).wait()
            pltpu.sync_copy(gbuf1, agg_sh.at[col_v.at[2 * j + 1]], add=True)
            return 0
        lax.fori_loop(0, NBLK // 2, _blk, 0)

        plsc.subcore_barrier()

        def _comb(i, _2):
            rs = i * CH
            pltpu.sync_copy(agg_sh.at[pl.ds(base + rs, CH)], abuf)
            pltpu.sync_copy(out_hbm.at[pl.ds(hrow + rs, CH)], ubuf)
            pltpu.sync_copy(g_hbm.at[pl.ds(hrow + rs, CH)], gchk)

            def _row(g, _3):
                cf16 = coef_v[pl.ds(rs + g * 16, 16)]
                for r in range(16):
                    cf = cf16[r]
                    rr = g * 16 + r
                    for f in range(NV):
                        ds16 = pl.ds(f * 16, 16)
                        ubuf[rr, ds16] = (
                            cf * (abuf[rr, ds16] + ubuf[rr, ds16])
                            + gchk[rr, ds16])
                return 0
            lax.fori_loop(0, CH // 16, _row, 0)

            pltpu.sync_copy(zbuf, agg_sh.at[pl.ds(base + rs, CH)])
            pltpu.sync_copy(ubuf, out_hbm.at[pl.ds(hrow + rs, CH)])
            return 0
        lax.fori_loop(0, NCH, _comb, 0)

        plsc.subcore_barrier()
        return 0
    lax.fori_loop(0, K, _step, 0)

    def _fin(i, _):
        rs = i * CH
        pltpu.sync_copy(out_hbm.at[pl.ds(hrow + rs, CH)], ubuf)

        def _sc16(g, _2):
            sq16 = sqd_v[pl.ds(rs + g * 16, 16)]
            for r in range(16):
                sq = sq16[r]
                for f in range(NV):
                    ds16 = pl.ds(f * 16, 16)
                    ubuf[g * 16 + r, ds16] = ubuf[g * 16 + r, ds16] * sq
            return 0
        lax.fori_loop(0, CH // 16, _sc16, 0)

        pltpu.sync_copy(ubuf, out_hbm.at[pl.ds(hrow + rs, CH)])
        return 0
    lax.fori_loop(0, NCH, _fin, 0)


def kernel(x, edge_index, W1, b1, W2, b2):
    ei = edge_index.astype(jnp.int32)
    row, col = ei[0], ei[1]
    npad = EP - E
    padnode = N + (jnp.arange(npad, dtype=jnp.int32) % (NP - N))
    rowp = jnp.concatenate([row, padnode])
    colp = jnp.concatenate([col, padnode])
    rowb = jnp.stack([rowp, rowp + NP]).reshape(2, NTILE, NBLK, B)
    colb = colp.reshape(NTILE, NBLK, B)

    h = _fc(x, W1, b1.reshape(1, D), relu=True)
    hp = jnp.pad(h, ((0, NP - N), (0, 0)))
    H = jnp.concatenate([hp[:, :F], hp[:, F:]], axis=0)

    zh, _ = _appnp_sc(H, rowb, colb)
    z = jnp.concatenate([zh[:N], zh[NP:NP + N]], axis=1)

    return _fc(z, W2, b2.reshape(1, D), relu=False)


# speedup vs baseline: 13.6617x; 1.2678x over previous
import functools

import jax
import jax.numpy as jnp
from jax import lax
from jax.experimental import pallas as pl
from jax.experimental.pallas import tpu as pltpu
from jax.experimental.pallas import tpu_sc as plsc

N = 10000
E = 320000
D = 128
K = 10
ALPHA = 0.1

NP = 10240
EP = 327680
NTILE = 16
ET = EP // NTILE
B = 128
NBLK = ET // B
NT = NP // NTILE
F = 64
NV = F // 16
CH = 64
NCH = NT // CH


def _fc_relu_body(x_ref, w_ref, b_ref, o_ref):
    o_ref[...] = jnp.maximum(
        jnp.dot(x_ref[...], w_ref[...], preferred_element_type=jnp.float32)
        + b_ref[...], 0.0)


def _fc_body(x_ref, w_ref, b_ref, o_ref):
    o_ref[...] = (
        jnp.dot(x_ref[...], w_ref[...], preferred_element_type=jnp.float32)
        + b_ref[...])


def _fc(x, w, b, relu):
    m = x.shape[0]
    blk = 1000
    return pl.pallas_call(
        _fc_relu_body if relu else _fc_body,
        grid=(m // blk,),
        in_specs=[
            pl.BlockSpec((blk, D), lambda i: (i, 0)),
            pl.BlockSpec((D, D), lambda i: (0, 0)),
            pl.BlockSpec((1, D), lambda i: (0, 0)),
        ],
        out_specs=pl.BlockSpec((blk, D), lambda i: (i, 0)),
        out_shape=jax.ShapeDtypeStruct((m, D), jnp.float32),
    )(x, w, b)


_mesh = plsc.VectorSubcoreMesh(core_axis_name="c", subcore_axis_name="s")


@functools.partial(
    pl.kernel,
    mesh=_mesh,
    compiler_params=pltpu.CompilerParams(use_tc_tiling_on_sc=False),
    out_type=(
        jax.ShapeDtypeStruct((2 * NP, F), jnp.float32),
        jax.ShapeDtypeStruct((2 * NP, F), jnp.float32),
    ),
    scratch_types=[
        pltpu.VMEM((NBLK, B), jnp.int32),
        pltpu.VMEM((NBLK, B), jnp.int32),
        pltpu.VMEM((B, F), jnp.float32),
        pltpu.VMEM((B, F), jnp.float32),
        pltpu.VMEM((CH, F), jnp.float32),
        pltpu.VMEM((CH, F), jnp.float32),
        pltpu.VMEM((CH, F), jnp.float32),
        pltpu.VMEM((CH, F), jnp.float32),
        pltpu.VMEM((NT,), jnp.float32),
        pltpu.VMEM((NT,), jnp.float32),
        pltpu.VMEM((NT,), jnp.float32),
        pltpu.VMEM((B,), jnp.float32),
        pltpu.VMEM_SHARED((NP, F), jnp.float32),
        pltpu.VMEM_SHARED((NP,), jnp.float32),
        pltpu.SemaphoreType.DMA,
        pltpu.SemaphoreType.DMA,
    ],
)
def _appnp_sc(h_hbm, rowb_hbm, col_hbm, out_hbm, g_hbm,
              row_v, col_v, gbuf0, gbuf1, abuf, ubuf, gchk, zbuf,
              coef_v, sqd_v, sv_v, ones_v, agg_sh, deg_sh, sem, sem_s):
    c = lax.axis_index("c")
    s = lax.axis_index("s")
    base = s * NT
    hrow = c * NP + base

    pltpu.sync_copy(rowb_hbm.at[c, s], row_v)
    pltpu.sync_copy(col_hbm.at[s], col_v)

    zf16 = jnp.zeros((16,), jnp.float32)

    def _zero_zbuf(i, _):
        r = i // NV
        f = (i % NV) * 16
        zbuf[r, pl.ds(f, 16)] = zf16
        return 0
    lax.fori_loop(0, CH * NV, _zero_zbuf, 0)

    def _zero_sv(i, _):
        sv_v[pl.ds(i * 16, 16)] = zf16
        return 0
    lax.fori_loop(0, NT // 16, _zero_sv, 0)

    def _ones(i, _):
        ones_v[pl.ds(i * 16, 16)] = jnp.ones((16,), jnp.float32)
        return 0
    lax.fori_loop(0, B // 16, _ones, 0)

    pltpu.sync_copy(sv_v, deg_sh.at[pl.ds(base, NT)])

    def _zero_agg(i, _):
        pltpu.sync_copy(zbuf, agg_sh.at[pl.ds(base + i * CH, CH)])
        return 0
    lax.fori_loop(0, NCH, _zero_agg, 0)

    plsc.subcore_barrier()

    def _deg_blk(j, _):
        pltpu.sync_copy(ones_v, deg_sh.at[col_v.at[j]], add=True)
        return 0
    lax.fori_loop(0, NBLK, _deg_blk, 0)

    plsc.subcore_barrier()

    pltpu.sync_copy(deg_sh.at[pl.ds(base, NT)], sv_v)

    def _coef(i, _):
        ds16 = pl.ds(i * 16, 16)
        d = sv_v[ds16] + 1.0
        y = 0.5 * (d + 1.0)
        for _it in range(20):
            y = 0.5 * (y + d / y)
        coef_v[ds16] = (1.0 - ALPHA) / d
        sqd_v[ds16] = y
        sv_v[ds16] = 1.0 / y
        return 0
    lax.fori_loop(0, NT // 16, _coef, 0)

    def _u0(i, _):
        rs = i * CH
        pltpu.sync_copy(h_hbm.at[pl.ds(hrow + rs, CH)], ubuf)

        def _sc16(g, _2):
            sv16 = sv_v[pl.ds(rs + g * 16, 16)]
            for r in range(16):
                sv = sv16[r]
                for f in range(NV):
                    ds16 = pl.ds(f * 16, 16)
                    val = ubuf[g * 16 + r, ds16] * sv
                    ubuf[g * 16 + r, ds16] = val
                    gchk[g * 16 + r, ds16] = val * ALPHA
            return 0
        lax.fori_loop(0, CH // 16, _sc16, 0)

        pltpu.sync_copy(ubuf, out_hbm.at[pl.ds(hrow + rs, CH)])
        pltpu.sync_copy(gchk, g_hbm.at[pl.ds(hrow + rs, CH)])
        return 0
    lax.fori_loop(0, NCH, _u0, 0)

    plsc.subcore_barrier()

    def _step(_k, _):
        pltpu.async_copy(out_hbm.at[row_v.at[0]], gbuf0, sem)

        def _blk(j, _2):
            a = 2 * j
            b = 2 * j + 1
            pltpu.make_async_copy(out_hbm.at[row_v.at[a]], gbuf0, sem).wait()

            @pl.when(j > 0)
            def _drain_prev():
                pltpu.make_async_copy(gbuf1, agg_sh.at[col_v.at[a]],
                                      sem_s).wait()

            pltpu.async_copy(out_hbm.at[row_v.at[b]], gbuf1, sem)
            pltpu.async_copy(gbuf0, agg_sh.at[col_v.at[a]], sem_s, add=True)
            pltpu.make_async_copy(out_hbm.at[row_v.at[b]], gbuf1, sem).wait()
            pltpu.make_async_copy(gbuf0, agg_sh.at[col_v.at[a]], sem_s).wait()

            @pl.when(j < NBLK // 2 - 1)
            def _next_gather():
                pltpu.async_copy(out_hbm.at[row_v.at[a + 2]], gbuf0, sem)

            pltpu.async_copy(gbuf1, agg_sh.at[col_v.at[b]], sem_s, add=True)
            return 0
        lax.fori_loop(0, NBLK // 2, _blk, 0)
        pltpu.make_async_copy(gbuf1, agg_sh.at[col_v.at[0]], sem_s).wait()

        plsc.subcore_barrier()

        def _comb(i, _2):
            rs = i * CH
            pltpu.sync_copy(agg_sh.at[pl.ds(base + rs, CH)], abuf)
            pltpu.sync_copy(out_hbm.at[pl.ds(hrow + rs, CH)], ubuf)
            pltpu.sync_copy(g_hbm.at[pl.ds(hrow + rs, CH)], gchk)

            def _row(g, _3):
                cf16 = coef_v[pl.ds(rs + g * 16, 16)]
                for r in range(16):
                    cf = cf16[r]
                    rr = g * 16 + r
                    for f in range(NV):
                        ds16 = pl.ds(f * 16, 16)
                        ubuf[rr, ds16] = (
                            cf * (abuf[rr, ds16] + ubuf[rr, ds16])
                            + gchk[rr, ds16])
                return 0
            lax.fori_loop(0, CH // 16, _row, 0)

            pltpu.sync_copy(zbuf, agg_sh.at[pl.ds(base + rs, CH)])
            pltpu.sync_copy(ubuf, out_hbm.at[pl.ds(hrow + rs, CH)])
            return 0
        lax.fori_loop(0, NCH, _comb, 0)

        plsc.subcore_barrier()
        return 0
    lax.fori_loop(0, K, _step, 0)

    def _fin(i, _):
        rs = i * CH
        pltpu.sync_copy(out_hbm.at[pl.ds(hrow + rs, CH)], ubuf)

        def _sc16(g, _2):
            sq16 = sqd_v[pl.ds(rs + g * 16, 16)]
            for r in range(16):
                sq = sq16[r]
                for f in range(NV):
                    ds16 = pl.ds(f * 16, 16)
                    ubuf[g * 16 + r, ds16] = ubuf[g * 16 + r, ds16] * sq
            return 0
        lax.fori_loop(0, CH // 16, _sc16, 0)

        pltpu.sync_copy(ubuf, out_hbm.at[pl.ds(hrow + rs, CH)])
        return 0
    lax.fori_loop(0, NCH, _fin, 0)


def kernel(x, edge_index, W1, b1, W2, b2):
    ei = edge_index.astype(jnp.int32)
    row, col = ei[0], ei[1]
    npad = EP - E
    padnode = N + (jnp.arange(npad, dtype=jnp.int32) % (NP - N))
    rowp = jnp.concatenate([row, padnode])
    colp = jnp.concatenate([col, padnode])
    rowb = jnp.stack([rowp, rowp + NP]).reshape(2, NTILE, NBLK, B)
    colb = colp.reshape(NTILE, NBLK, B)

    h = _fc(x, W1, b1.reshape(1, D), relu=True)
    hp = jnp.pad(h, ((0, NP - N), (0, 0)))
    H = jnp.concatenate([hp[:, :F], hp[:, F:]], axis=0)

    zh, _ = _appnp_sc(H, rowb, colb)
    z = jnp.concatenate([zh[:N], zh[NP:NP + N]], axis=1)

    return _fc(z, W2, b2.reshape(1, D), relu=False)


# speedup vs baseline: 19.8546x; 1.4533x over previous
import functools

import jax
import jax.numpy as jnp
from jax import lax
from jax.experimental import pallas as pl
from jax.experimental.pallas import tpu as pltpu
from jax.experimental.pallas import tpu_sc as plsc

N = 10000
E = 320000
D = 128
K = 10
ALPHA = 0.1

NP = 10240
EP = 327680
NTILE = 16
ET = EP // NTILE
B = 128
NBLK = ET // B
NT = NP // NTILE
F = 64
NV = F // 16
CH = 64
NCH = NT // CH


def _fc_relu_body(x_ref, w_ref, b_ref, o_ref):
    o_ref[...] = jnp.maximum(
        jnp.dot(x_ref[...], w_ref[...], preferred_element_type=jnp.float32)
        + b_ref[...], 0.0)


def _fc_body(x_ref, w_ref, b_ref, o_ref):
    o_ref[...] = (
        jnp.dot(x_ref[...], w_ref[...], preferred_element_type=jnp.float32)
        + b_ref[...])


def _fc(x, w, b, relu):
    m = x.shape[0]
    blk = 1000
    return pl.pallas_call(
        _fc_relu_body if relu else _fc_body,
        grid=(m // blk,),
        in_specs=[
            pl.BlockSpec((blk, D), lambda i: (i, 0)),
            pl.BlockSpec((D, D), lambda i: (0, 0)),
            pl.BlockSpec((1, D), lambda i: (0, 0)),
        ],
        out_specs=pl.BlockSpec((blk, D), lambda i: (i, 0)),
        out_shape=jax.ShapeDtypeStruct((m, D), jnp.float32),
    )(x, w, b)


_mesh = plsc.VectorSubcoreMesh(core_axis_name="c", subcore_axis_name="s")


@functools.partial(
    pl.kernel,
    mesh=_mesh,
    compiler_params=pltpu.CompilerParams(use_tc_tiling_on_sc=False),
    out_type=(
        jax.ShapeDtypeStruct((2 * NP, F), jnp.float32),
        jax.ShapeDtypeStruct((2 * NP, F), jnp.float32),
    ),
    scratch_types=[
        pltpu.VMEM((NBLK, B), jnp.int32),
        pltpu.VMEM((NBLK, B), jnp.int32),
        pltpu.VMEM((4, B, F), jnp.float32),
        pltpu.VMEM((CH, F), jnp.float32),
        pltpu.VMEM((CH, F), jnp.float32),
        pltpu.VMEM((CH, F), jnp.float32),
        pltpu.VMEM((NT,), jnp.float32),
        pltpu.VMEM((NT,), jnp.float32),
        pltpu.VMEM((NT,), jnp.float32),
        pltpu.VMEM((B,), jnp.float32),
        pltpu.VMEM_SHARED((NP, F), jnp.float32),
        pltpu.VMEM_SHARED((NP,), jnp.float32),
        pltpu.SemaphoreType.DMA,
        ((pltpu.SemaphoreType.DMA,) * 4),
        ((pltpu.SemaphoreType.DMA,) * 4),
    ],
)
def _appnp_sc(h_hbm, rowb_hbm, col_hbm, out_hbm, g_hbm,
              row_v, col_v, gring, abuf, ubuf, gchk,
              coef_v, sqd_v, sv_v, ones_v, agg_sh, deg_sh, sem,
              sem_g, sem_s):
    c = lax.axis_index("c")
    s = lax.axis_index("s")
    base = s * NT
    hrow = c * NP + base

    pltpu.sync_copy(rowb_hbm.at[c, s], row_v)
    pltpu.sync_copy(col_hbm.at[s], col_v)

    zf16 = jnp.zeros((16,), jnp.float32)

    def _zero_abuf(i, _):
        r = i // NV
        f = (i % NV) * 16
        abuf[r, pl.ds(f, 16)] = zf16
        return 0
    lax.fori_loop(0, CH * NV, _zero_abuf, 0)

    def _zero_sv(i, _):
        sv_v[pl.ds(i * 16, 16)] = zf16
        return 0
    lax.fori_loop(0, NT // 16, _zero_sv, 0)

    def _ones(i, _):
        ones_v[pl.ds(i * 16, 16)] = jnp.ones((16,), jnp.float32)
        return 0
    lax.fori_loop(0, B // 16, _ones, 0)

    pltpu.sync_copy(sv_v, deg_sh.at[pl.ds(base, NT)])

    def _zero_agg(i, _):
        pltpu.sync_copy(abuf, agg_sh.at[pl.ds(base + i * CH, CH)])
        return 0
    lax.fori_loop(0, NCH, _zero_agg, 0)

    plsc.subcore_barrier()

    def _deg_blk(j, _):
        pltpu.sync_copy(ones_v, deg_sh.at[col_v.at[j]], add=True)
        return 0
    lax.fori_loop(0, NBLK, _deg_blk, 0)

    plsc.subcore_barrier()

    pltpu.sync_copy(deg_sh.at[pl.ds(base, NT)], sv_v)

    def _coef(i, _):
        ds16 = pl.ds(i * 16, 16)
        d = sv_v[ds16] + 1.0
        y = 0.5 * (d + 1.0)
        for _it in range(20):
            y = 0.5 * (y + d / y)
        coef_v[ds16] = (1.0 - ALPHA) / d
        sqd_v[ds16] = y
        sv_v[ds16] = 1.0 / y
        return 0
    lax.fori_loop(0, NT // 16, _coef, 0)

    def _u0(i, _):
        rs = i * CH
        pltpu.sync_copy(h_hbm.at[pl.ds(hrow + rs, CH)], ubuf)

        def _sc16(g, _2):
            sv16 = sv_v[pl.ds(rs + g * 16, 16)]
            for r in range(16):
                sv = sv16[r]
                for f in range(NV):
                    ds16 = pl.ds(f * 16, 16)
                    val = ubuf[g * 16 + r, ds16] * sv
                    ubuf[g * 16 + r, ds16] = val
                    gchk[g * 16 + r, ds16] = val * ALPHA
            return 0
        lax.fori_loop(0, CH // 16, _sc16, 0)

        pltpu.sync_copy(ubuf, out_hbm.at[pl.ds(hrow + rs, CH)])
        pltpu.sync_copy(gchk, g_hbm.at[pl.ds(hrow + rs, CH)])
        return 0
    lax.fori_loop(0, NCH, _u0, 0)

    plsc.subcore_barrier()

    def _step(_k, _):
        for m in range(4):
            pltpu.async_copy(out_hbm.at[row_v.at[m]], gring.at[m], sem_g[m])

        def _blk(j, _2):
            for m in range(4):
                a = 4 * j + m
                pltpu.make_async_copy(out_hbm.at[row_v.at[a]],
                                      gring.at[m], sem_g[m]).wait()
                pltpu.async_copy(gring.at[m], agg_sh.at[col_v.at[a]],
                                 sem_s[m], add=True)
            for m in range(4):
                a = 4 * j + m
                pltpu.make_async_copy(gring.at[m], agg_sh.at[col_v.at[a]],
                                      sem_s[m]).wait()

                @pl.when(j < NBLK // 4 - 1)
                def _next_gather():
                    pltpu.async_copy(out_hbm.at[row_v.at[a + 4]],
                                     gring.at[m], sem_g[m])
            return 0
        lax.fori_loop(0, NBLK // 4, _blk, 0)

        plsc.subcore_barrier()

        def _comb(i, _2):
            rs = i * CH
            pltpu.sync_copy(agg_sh.at[pl.ds(base + rs, CH)], abuf)
            pltpu.sync_copy(out_hbm.at[pl.ds(hrow + rs, CH)], ubuf)
            pltpu.sync_copy(g_hbm.at[pl.ds(hrow + rs, CH)], gchk)

            def _row(g, _3):
                cf16 = coef_v[pl.ds(rs + g * 16, 16)]
                for r in range(16):
                    cf = cf16[r]
                    rr = g * 16 + r
                    for f in range(NV):
                        ds16 = pl.ds(f * 16, 16)
                        ubuf[rr, ds16] = (
                            cf * (abuf[rr, ds16] + ubuf[rr, ds16])
                            + gchk[rr, ds16])
                        abuf[rr, ds16] = zf16
                return 0
            lax.fori_loop(0, CH // 16, _row, 0)

            pltpu.sync_copy(abuf, agg_sh.at[pl.ds(base + rs, CH)])
            pltpu.sync_copy(ubuf, out_hbm.at[pl.ds(hrow + rs, CH)])
            return 0
        lax.fori_loop(0, NCH, _comb, 0)

        plsc.subcore_barrier()
        return 0
    lax.fori_loop(0, K, _step, 0)

    def _fin(i, _):
        rs = i * CH
        pltpu.sync_copy(out_hbm.at[pl.ds(hrow + rs, CH)], ubuf)

        def _sc16(g, _2):
            sq16 = sqd_v[pl.ds(rs + g * 16, 16)]
            for r in range(16):
                sq = sq16[r]
                for f in range(NV):
                    ds16 = pl.ds(f * 16, 16)
                    ubuf[g * 16 + r, ds16] = ubuf[g * 16 + r, ds16] * sq
            return 0
        lax.fori_loop(0, CH // 16, _sc16, 0)

        pltpu.sync_copy(ubuf, out_hbm.at[pl.ds(hrow + rs, CH)])
        return 0
    lax.fori_loop(0, NCH, _fin, 0)


def kernel(x, edge_index, W1, b1, W2, b2):
    ei = edge_index.astype(jnp.int32)
    row, col = ei[0], ei[1]
    npad = EP - E
    padnode = N + (jnp.arange(npad, dtype=jnp.int32) % (NP - N))
    rowp = jnp.concatenate([row, padnode])
    colp = jnp.concatenate([col, padnode])
    rowb = jnp.stack([rowp, rowp + NP]).reshape(2, NTILE, NBLK, B)
    colb = colp.reshape(NTILE, NBLK, B)

    h = _fc(x, W1, b1.reshape(1, D), relu=True)
    hp = jnp.pad(h, ((0, NP - N), (0, 0)))
    H = jnp.concatenate([hp[:, :F], hp[:, F:]], axis=0)

    zh, _ = _appnp_sc(H, rowb, colb)
    z = jnp.concatenate([zh[:N], zh[NP:NP + N]], axis=1)

    return _fc(z, W2, b2.reshape(1, D), relu=False)


# speedup vs baseline: 20.5369x; 1.0344x over previous
import functools

import jax
import jax.numpy as jnp
from jax import lax
from jax.experimental import pallas as pl
from jax.experimental.pallas import tpu as pltpu
from jax.experimental.pallas import tpu_sc as plsc

N = 10000
E = 320000
D = 128
K = 10
ALPHA = 0.1

NP = 10240
EP = 327680
NTILE = 16
ET = EP // NTILE
B = 128
NBLK = ET // B
NT = NP // NTILE
F = 64
NV = F // 16
CH = 32
NCH = NT // CH


def _fc_relu_body(x_ref, w_ref, b_ref, o_ref):
    o_ref[...] = jnp.maximum(
        jnp.dot(x_ref[...], w_ref[...], preferred_element_type=jnp.float32)
        + b_ref[...], 0.0)


def _fc_body(x_ref, w_ref, b_ref, o_ref):
    o_ref[...] = (
        jnp.dot(x_ref[...], w_ref[...], preferred_element_type=jnp.float32)
        + b_ref[...])


def _fc(x, w, b, relu):
    m = x.shape[0]
    blk = 1000
    return pl.pallas_call(
        _fc_relu_body if relu else _fc_body,
        grid=(m // blk,),
        in_specs=[
            pl.BlockSpec((blk, D), lambda i: (i, 0)),
            pl.BlockSpec((D, D), lambda i: (0, 0)),
            pl.BlockSpec((1, D), lambda i: (0, 0)),
        ],
        out_specs=pl.BlockSpec((blk, D), lambda i: (i, 0)),
        out_shape=jax.ShapeDtypeStruct((m, D), jnp.float32),
    )(x, w, b)


_mesh = plsc.VectorSubcoreMesh(core_axis_name="c", subcore_axis_name="s")


@functools.partial(
    pl.kernel,
    mesh=_mesh,
    compiler_params=pltpu.CompilerParams(use_tc_tiling_on_sc=False),
    out_type=(
        jax.ShapeDtypeStruct((2 * NP, F), jnp.float32),
        jax.ShapeDtypeStruct((2 * NP, F), jnp.float32),
    ),
    scratch_types=[
        pltpu.VMEM((NBLK, B), jnp.int32),
        pltpu.VMEM((NBLK, B), jnp.int32),
        pltpu.VMEM((4, B, F), jnp.float32),
        pltpu.VMEM((2, CH, F), jnp.float32),
        pltpu.VMEM((2, CH, F), jnp.float32),
        pltpu.VMEM((2, CH, F), jnp.float32),
        pltpu.VMEM((NT,), jnp.float32),
        pltpu.VMEM((NT,), jnp.float32),
        pltpu.VMEM((NT,), jnp.float32),
        pltpu.VMEM((B,), jnp.float32),
        pltpu.VMEM_SHARED((NP, F), jnp.float32),
        pltpu.VMEM_SHARED((NP,), jnp.float32),
        ((pltpu.SemaphoreType.DMA,) * 4),
        ((pltpu.SemaphoreType.DMA,) * 4),
        ((pltpu.SemaphoreType.DMA,) * 2),
        ((pltpu.SemaphoreType.DMA,) * 2),
    ],
)
def _appnp_sc(h_hbm, rowb_hbm, col_hbm, out_hbm, g_hbm,
              row_v, col_v, gring, abuf2, ubuf2, gchk2,
              coef_v, sqd_v, sv_v, ones_v, agg_sh, deg_sh,
              sem_g, sem_s, sem_r, sem_w):
    c = lax.axis_index("c")
    s = lax.axis_index("s")
    base = s * NT
    hrow = c * NP + base

    zf16 = jnp.zeros((16,), jnp.float32)

    pltpu.sync_copy(rowb_hbm.at[c, s], row_v)
    pltpu.sync_copy(col_hbm.at[s], col_v)

    def _zero_abuf(i, _):
        r = i // NV
        f = (i % NV) * 16
        abuf2[0, r, pl.ds(f, 16)] = zf16
        return 0
    lax.fori_loop(0, CH * NV, _zero_abuf, 0)

    def _zero_sv(i, _):
        sv_v[pl.ds(i * 16, 16)] = zf16
        return 0
    lax.fori_loop(0, NT // 16, _zero_sv, 0)

    def _ones(i, _):
        ones_v[pl.ds(i * 16, 16)] = jnp.ones((16,), jnp.float32)
        return 0
    lax.fori_loop(0, B // 16, _ones, 0)

    pltpu.sync_copy(sv_v, deg_sh.at[pl.ds(base, NT)])

    def _zero_agg(i, _):
        pltpu.sync_copy(abuf2.at[0], agg_sh.at[pl.ds(base + i * CH, CH)])
        return 0
    lax.fori_loop(0, NCH, _zero_agg, 0)

    plsc.subcore_barrier()

    def _deg_blk(j, _):
        pltpu.sync_copy(ones_v, deg_sh.at[col_v.at[j]], add=True)
        return 0
    lax.fori_loop(0, NBLK, _deg_blk, 0)

    plsc.subcore_barrier()

    pltpu.sync_copy(deg_sh.at[pl.ds(base, NT)], sv_v)

    def _coef(i, _):
        ds16 = pl.ds(i * 16, 16)
        d = sv_v[ds16] + 1.0
        y = 0.5 * (d + 1.0)
        for _it in range(20):
            y = 0.5 * (y + d / y)
        coef_v[ds16] = (1.0 - ALPHA) / d
        sqd_v[ds16] = y
        sv_v[ds16] = 1.0 / y
        return 0
    lax.fori_loop(0, NT // 16, _coef, 0)

    def _u0(i, _):
        rs = i * CH
        pltpu.sync_copy(h_hbm.at[pl.ds(hrow + rs, CH)], ubuf2.at[0])

        def _sc16(g, _2):
            sv16 = sv_v[pl.ds(rs + g * 16, 16)]
            for r in range(16):
                sv = sv16[r]
                for f in range(NV):
                    ds16 = pl.ds(f * 16, 16)
                    val = ubuf2[0, g * 16 + r, ds16] * sv
                    ubuf2[0, g * 16 + r, ds16] = val
                    gchk2[0, g * 16 + r, ds16] = val * ALPHA
            return 0
        lax.fori_loop(0, CH // 16, _sc16, 0)

        pltpu.sync_copy(ubuf2.at[0], out_hbm.at[pl.ds(hrow + rs, CH)])
        pltpu.sync_copy(gchk2.at[0], g_hbm.at[pl.ds(hrow + rs, CH)])
        return 0
    lax.fori_loop(0, NCH, _u0, 0)

    plsc.subcore_barrier()

    def _issue_r(i, st):
        rs = i * CH
        pltpu.async_copy(out_hbm.at[pl.ds(hrow + rs, CH)], ubuf2.at[st],
                         sem_r[st])
        pltpu.async_copy(g_hbm.at[pl.ds(hrow + rs, CH)], gchk2.at[st],
                         sem_r[st])

    def _wait_r(i, st):
        rs = i * CH
        pltpu.make_async_copy(out_hbm.at[pl.ds(hrow + rs, CH)], ubuf2.at[st],
                              sem_r[st]).wait()
        pltpu.make_async_copy(g_hbm.at[pl.ds(hrow + rs, CH)], gchk2.at[st],
                              sem_r[st]).wait()

    def _issue_w(i, st):
        rs = i * CH
        pltpu.async_copy(ubuf2.at[st], out_hbm.at[pl.ds(hrow + rs, CH)],
                         sem_w[st])

    def _wait_w(i, st):
        rs = i * CH
        pltpu.make_async_copy(ubuf2.at[st], out_hbm.at[pl.ds(hrow + rs, CH)],
                              sem_w[st]).wait()

    def _compute(i, st):
        rs = i * CH
        pltpu.sync_copy(agg_sh.at[pl.ds(base + rs, CH)], abuf2.at[st])

        def _row(g, _3):
            cf16 = coef_v[pl.ds(rs + g * 16, 16)]
            for r in range(16):
                cf = cf16[r]
                rr = g * 16 + r
                for f in range(NV):
                    ds16 = pl.ds(f * 16, 16)
                    ubuf2[st, rr, ds16] = (
                        cf * (abuf2[st, rr, ds16] + ubuf2[st, rr, ds16])
                        + gchk2[st, rr, ds16])
                    abuf2[st, rr, ds16] = zf16
            return 0
        lax.fori_loop(0, CH // 16, _row, 0)

        pltpu.sync_copy(abuf2.at[st], agg_sh.at[pl.ds(base + rs, CH)])

    def _step(_k, _):
        for m in range(4):
            pltpu.async_copy(out_hbm.at[row_v.at[m]], gring.at[m], sem_g[m])

        def _blk(j, _2):
            for m in range(4):
                a = 4 * j + m
                pltpu.make_async_copy(out_hbm.at[row_v.at[a]],
                                      gring.at[m], sem_g[m]).wait()
                pltpu.async_copy(gring.at[m], agg_sh.at[col_v.at[a]],
                                 sem_s[m], add=True)
            for m in range(4):
                a = 4 * j + m
                pltpu.make_async_copy(gring.at[m], agg_sh.at[col_v.at[a]],
                                      sem_s[m]).wait()

                @pl.when(j < NBLK // 4 - 1)
                def _next_gather():
                    pltpu.async_copy(out_hbm.at[row_v.at[a + 4]],
                                     gring.at[m], sem_g[m])
            return 0
        lax.fori_loop(0, NBLK // 4, _blk, 0)

        plsc.subcore_barrier()

        _issue_r(0, 0)

        def _citer(i, _2):
            a = 2 * i
            b = 2 * i + 1
            _wait_r(a, 0)

            @pl.when(i > 0)
            def _ww1():
                _wait_w(b - 2, 1)
            _issue_r(b, 1)
            _compute(a, 0)
            _issue_w(a, 0)
            _wait_r(b, 1)
            _compute(b, 1)
            _issue_w(b, 1)

            @pl.when(i < NCH // 2 - 1)
            def _nr():
                _wait_w(a, 0)
                _issue_r(a + 2, 0)
            return 0
        lax.fori_loop(0, NCH // 2, _citer, 0)
        _wait_w(NCH - 2, 0)
        _wait_w(NCH - 1, 1)

        plsc.subcore_barrier()
        return 0
    lax.fori_loop(0, K, _step, 0)

    def _fin(i, _):
        rs = i * CH
        pltpu.sync_copy(out_hbm.at[pl.ds(hrow + rs, CH)], ubuf2.at[0])

        def _sc16(g, _2):
            sq16 = sqd_v[pl.ds(rs + g * 16, 16)]
            for r in range(16):
                sq = sq16[r]
                for f in range(NV):
                    ds16 = pl.ds(f * 16, 16)
                    ubuf2[0, g * 16 + r, ds16] = ubuf2[0, g * 16 + r, ds16] * sq
            return 0
        lax.fori_loop(0, CH // 16, _sc16, 0)

        pltpu.sync_copy(ubuf2.at[0], out_hbm.at[pl.ds(hrow + rs, CH)])
        return 0
    lax.fori_loop(0, NCH, _fin, 0)


def kernel(x, edge_index, W1, b1, W2, b2):
    ei = edge_index.astype(jnp.int32)
    row, col = ei[0], ei[1]
    npad = EP - E
    padnode = N + (jnp.arange(npad, dtype=jnp.int32) % (NP - N))
    rowp_ = jnp.concatenate([row, padnode])
    colp_ = jnp.concatenate([col, padnode])
    rowb = jnp.stack([rowp_, rowp_ + NP]).reshape(2, NTILE, NBLK, B)
    colb = colp_.reshape(NTILE, NBLK, B)

    h = _fc(x, W1, b1.reshape(1, D), relu=True)
    hp = jnp.pad(h, ((0, NP - N), (0, 0)))
    H = jnp.concatenate([hp[:, :F], hp[:, F:]], axis=0)

    zh, _ = _appnp_sc(H, rowb, colb)
    z = jnp.concatenate([zh[:N], zh[NP:NP + N]], axis=1)

    return _fc(z, W2, b2.reshape(1, D), relu=False)


# speedup vs baseline: 20.7564x; 1.0107x over previous
import functools

import jax
import jax.numpy as jnp
from jax import lax
from jax.experimental import pallas as pl
from jax.experimental.pallas import tpu as pltpu
from jax.experimental.pallas import tpu_sc as plsc

N = 10000
E = 320000
D = 128
K = 10
ALPHA = 0.1

NP = 10240
EP = 327680
NTILE = 16
ET = EP // NTILE
B = 128
NBLK = ET // B
RD = 8
NPAN = NBLK // RD
NT = NP // NTILE
F = 64
NV = F // 16
CH = 32
NCH = NT // CH


def _fc_relu_body(x_ref, w_ref, b_ref, o_ref):
    o_ref[...] = jnp.maximum(
        jnp.dot(x_ref[...], w_ref[...], preferred_element_type=jnp.float32)
        + b_ref[...], 0.0)


def _fc_body(x_ref, w_ref, b_ref, o_ref):
    o_ref[...] = (
        jnp.dot(x_ref[...], w_ref[...], preferred_element_type=jnp.float32)
        + b_ref[...])


def _fc(x, w, b, relu):
    m = x.shape[0]
    blk = 1000
    return pl.pallas_call(
        _fc_relu_body if relu else _fc_body,
        grid=(m // blk,),
        in_specs=[
            pl.BlockSpec((blk, D), lambda i: (i, 0)),
            pl.BlockSpec((D, D), lambda i: (0, 0)),
            pl.BlockSpec((1, D), lambda i: (0, 0)),
        ],
        out_specs=pl.BlockSpec((blk, D), lambda i: (i, 0)),
        out_shape=jax.ShapeDtypeStruct((m, D), jnp.float32),
    )(x, w, b)


_mesh = plsc.VectorSubcoreMesh(core_axis_name="c", subcore_axis_name="s")


@functools.partial(
    pl.kernel,
    mesh=_mesh,
    compiler_params=pltpu.CompilerParams(use_tc_tiling_on_sc=False),
    out_type=(
        jax.ShapeDtypeStruct((2 * NP, F), jnp.float32),
        jax.ShapeDtypeStruct((2 * NP, F), jnp.float32),
    ),
    scratch_types=[
        pltpu.VMEM((2 * RD, B), jnp.int32),
        pltpu.VMEM((2 * RD, B), jnp.int32),
        pltpu.VMEM((RD, B, F), jnp.float32),
        pltpu.VMEM((2, CH, F), jnp.float32),
        pltpu.VMEM((2, CH, F), jnp.float32),
        pltpu.VMEM((2, CH, F), jnp.float32),
        pltpu.VMEM((NT,), jnp.float32),
        pltpu.VMEM((NT,), jnp.float32),
        pltpu.VMEM((NT,), jnp.float32),
        pltpu.VMEM((B,), jnp.float32),
        pltpu.VMEM_SHARED((NP, F), jnp.float32),
        pltpu.VMEM_SHARED((NP,), jnp.float32),
        pltpu.SemaphoreType.DMA,
        ((pltpu.SemaphoreType.DMA,) * RD),
        ((pltpu.SemaphoreType.DMA,) * RD),
        ((pltpu.SemaphoreType.DMA,) * 2),
        ((pltpu.SemaphoreType.DMA,) * 2),
    ],
)
def _appnp_sc(h_hbm, rowb_hbm, col_hbm, out_hbm, g_hbm,
              rowp, colp, gring, abuf2, ubuf2, gchk2,
              coef_v, sqd_v, sv_v, ones_v, agg_sh, deg_sh,
              sem_i, sem_g, sem_s, sem_r, sem_w):
    c = lax.axis_index("c")
    s = lax.axis_index("s")
    base = s * NT
    hrow = c * NP + base

    zf16 = jnp.zeros((16,), jnp.float32)

    def _zero_abuf(i, _):
        r = i // NV
        f = (i % NV) * 16
        abuf2[0, r, pl.ds(f, 16)] = zf16
        return 0
    lax.fori_loop(0, CH * NV, _zero_abuf, 0)

    def _zero_sv(i, _):
        sv_v[pl.ds(i * 16, 16)] = zf16
        return 0
    lax.fori_loop(0, NT // 16, _zero_sv, 0)

    def _ones(i, _):
        ones_v[pl.ds(i * 16, 16)] = jnp.ones((16,), jnp.float32)
        return 0
    lax.fori_loop(0, B // 16, _ones, 0)

    pltpu.sync_copy(sv_v, deg_sh.at[pl.ds(base, NT)])

    def _zero_agg(i, _):
        pltpu.sync_copy(abuf2.at[0], agg_sh.at[pl.ds(base + i * CH, CH)])
        return 0
    lax.fori_loop(0, NCH, _zero_agg, 0)

    plsc.subcore_barrier()

    def _deg_pan(p, _):
        pltpu.sync_copy(col_hbm.at[s, pl.ds(p * RD, RD)],
                        colp.at[pl.ds(0, RD)])

        def _deg_blk(jj, _2):
            pltpu.sync_copy(ones_v, deg_sh.at[colp.at[jj]], add=True)
            return 0
        lax.fori_loop(0, RD, _deg_blk, 0)
        return 0
    lax.fori_loop(0, NPAN, _deg_pan, 0)

    plsc.subcore_barrier()

    pltpu.sync_copy(deg_sh.at[pl.ds(base, NT)], sv_v)

    def _coef(i, _):
        ds16 = pl.ds(i * 16, 16)
        d = sv_v[ds16] + 1.0
        y = 0.5 * (d + 1.0)
        for _it in range(20):
            y = 0.5 * (y + d / y)
        coef_v[ds16] = (1.0 - ALPHA) / d
        sqd_v[ds16] = y
        sv_v[ds16] = 1.0 / y
        return 0
    lax.fori_loop(0, NT // 16, _coef, 0)

    def _u0(i, _):
        rs = i * CH
        pltpu.sync_copy(h_hbm.at[pl.ds(hrow + rs, CH)], ubuf2.at[0])

        def _sc16(g, _2):
            sv16 = sv_v[pl.ds(rs + g * 16, 16)]
            for r in range(16):
                sv = sv16[r]
                for f in range(NV):
                    ds16 = pl.ds(f * 16, 16)
                    val = ubuf2[0, g * 16 + r, ds16] * sv
                    ubuf2[0, g * 16 + r, ds16] = val
                    gchk2[0, g * 16 + r, ds16] = val * ALPHA
            return 0
        lax.fori_loop(0, CH // 16, _sc16, 0)

        pltpu.sync_copy(ubuf2.at[0], out_hbm.at[pl.ds(hrow + rs, CH)])
        pltpu.sync_copy(gchk2.at[0], g_hbm.at[pl.ds(hrow + rs, CH)])
        return 0
    lax.fori_loop(0, NCH, _u0, 0)

    plsc.subcore_barrier()

    def _issue_r(i, st):
        rs = i * CH
        pltpu.async_copy(out_hbm.at[pl.ds(hrow + rs, CH)], ubuf2.at[st],
                         sem_r[st])
        pltpu.async_copy(g_hbm.at[pl.ds(hrow + rs, CH)], gchk2.at[st],
                         sem_r[st])

    def _wait_r(i, st):
        rs = i * CH
        pltpu.make_async_copy(out_hbm.at[pl.ds(hrow + rs, CH)], ubuf2.at[st],
                              sem_r[st]).wait()
        pltpu.make_async_copy(g_hbm.at[pl.ds(hrow + rs, CH)], gchk2.at[st],
                              sem_r[st]).wait()

    def _issue_w(i, st):
        rs = i * CH
        pltpu.async_copy(ubuf2.at[st], out_hbm.at[pl.ds(hrow + rs, CH)],
                         sem_w[st])

    def _wait_w(i, st):
        rs = i * CH
        pltpu.make_async_copy(ubuf2.at[st], out_hbm.at[pl.ds(hrow + rs, CH)],
                              sem_w[st]).wait()

    def _compute(i, st):
        rs = i * CH
        pltpu.sync_copy(agg_sh.at[pl.ds(base + rs, CH)], abuf2.at[st])

        def _row(g, _3):
            cf16 = coef_v[pl.ds(rs + g * 16, 16)]
            for r in range(16):
                cf = cf16[r]
                rr = g * 16 + r
                for f in range(NV):
                    ds16 = pl.ds(f * 16, 16)
                    ubuf2[st, rr, ds16] = (
                        cf * (abuf2[st, rr, ds16] + ubuf2[st, rr, ds16])
                        + gchk2[st, rr, ds16])
                    abuf2[st, rr, ds16] = zf16
            return 0
        lax.fori_loop(0, CH // 16, _row, 0)

        pltpu.sync_copy(abuf2.at[st], agg_sh.at[pl.ds(base + rs, CH)])

    def _step(_k, _):
        pltpu.sync_copy(rowb_hbm.at[c, s, pl.ds(0, RD)],
                        rowp.at[pl.ds(0, RD)])
        pltpu.sync_copy(col_hbm.at[s, pl.ds(0, RD)],
                        colp.at[pl.ds(0, RD)])
        for m in range(RD):
            pltpu.async_copy(out_hbm.at[rowp.at[m]], gring.at[m], sem_g[m])

        def _pan(p, _2):
            par = lax.rem(p, 2)
            ib = par * RD
            nb = RD - ib

            @pl.when(p + 1 < NPAN)
            def _pf():
                pltpu.async_copy(
                    rowb_hbm.at[c, s, pl.ds((p + 1) * RD, RD)],
                    rowp.at[pl.ds(nb, RD)], sem_i)
                pltpu.async_copy(
                    col_hbm.at[s, pl.ds((p + 1) * RD, RD)],
                    colp.at[pl.ds(nb, RD)], sem_i)

            for m in range(RD):
                pltpu.make_async_copy(out_hbm.at[rowp.at[ib + m]],
                                      gring.at[m], sem_g[m]).wait()
                pltpu.async_copy(gring.at[m], agg_sh.at[colp.at[ib + m]],
                                 sem_s[m], add=True)

            @pl.when(p + 1 < NPAN)
            def _wi():
                pltpu.make_async_copy(
                    rowb_hbm.at[c, s, pl.ds(0, RD)],
                    rowp.at[pl.ds(nb, RD)], sem_i).wait()
                pltpu.make_async_copy(
                    col_hbm.at[s, pl.ds(0, RD)],
                    colp.at[pl.ds(nb, RD)], sem_i).wait()

            for m in range(RD):
                pltpu.make_async_copy(gring.at[m],
                                      agg_sh.at[colp.at[ib + m]],
                                      sem_s[m]).wait()

                @pl.when(p + 1 < NPAN)
                def _ng(m=m):
                    pltpu.async_copy(out_hbm.at[rowp.at[nb + m]],
                                     gring.at[m], sem_g[m])
            return 0
        lax.fori_loop(0, NPAN, _pan, 0)

        plsc.subcore_barrier()

        _issue_r(0, 0)

        def _citer(i, _2):
            a = 2 * i
            b = 2 * i + 1
            _wait_r(a, 0)

            @pl.when(i > 0)
            def _ww1():
                _wait_w(b - 2, 1)
            _issue_r(b, 1)
            _compute(a, 0)
            _issue_w(a, 0)
            _wait_r(b, 1)
            _compute(b, 1)
            _issue_w(b, 1)

            @pl.when(i < NCH // 2 - 1)
            def _nr():
                _wait_w(a, 0)
                _issue_r(a + 2, 0)
            return 0
        lax.fori_loop(0, NCH // 2, _citer, 0)
        _wait_w(NCH - 2, 0)
        _wait_w(NCH - 1, 1)

        plsc.subcore_barrier()
        return 0
    lax.fori_loop(0, K, _step, 0)

    def _fin(i, _):
        rs = i * CH
        pltpu.sync_copy(out_hbm.at[pl.ds(hrow + rs, CH)], ubuf2.at[0])

        def _sc16(g, _2):
            sq16 = sqd_v[pl.ds(rs + g * 16, 16)]
            for r in range(16):
                sq = sq16[r]
                for f in range(NV):
                    ds16 = pl.ds(f * 16, 16)
                    ubuf2[0, g * 16 + r, ds16] = ubuf2[0, g * 16 + r, ds16] * sq
            return 0
        lax.fori_loop(0, CH // 16, _sc16, 0)

        pltpu.sync_copy(ubuf2.at[0], out_hbm.at[pl.ds(hrow + rs, CH)])
        return 0
    lax.fori_loop(0, NCH, _fin, 0)


def kernel(x, edge_index, W1, b1, W2, b2):
    ei = edge_index.astype(jnp.int32)
    row, col = ei[0], ei[1]
    npad = EP - E
    padnode = N + (jnp.arange(npad, dtype=jnp.int32) % (NP - N))
    rowp_ = jnp.concatenate([row, padnode])
    colp_ = jnp.concatenate([col, padnode])
    rowb = jnp.stack([rowp_, rowp_ + NP]).reshape(2, NTILE, NBLK, B)
    colb = colp_.reshape(NTILE, NBLK, B)

    h = _fc(x, W1, b1.reshape(1, D), relu=True)
    hp = jnp.pad(h, ((0, NP - N), (0, 0)))
    H = jnp.concatenate([hp[:, :F], hp[:, F:]], axis=0)

    zh, _ = _appnp_sc(H, rowb, colb)
    z = jnp.concatenate([zh[:N], zh[NP:NP + N]], axis=1)

    return _fc(z, W2, b2.reshape(1, D), relu=False)
